# baseline (speedup 1.0000x reference)
import functools
import jax
import jax.numpy as jnp
from jax import lax
from jax.experimental import pallas as pl
from jax.experimental.pallas import tpu as pltpu
from jax.experimental.pallas import tpu_sc as plsc

B = 16
N_IN = 100000
N_OUT = 10000
FS = 512

NC = 2
NS = 16
NW = NC * NS
NSEG_PAD = 10240
CPW = NSEG_PAD // NW
CH = 512
GSUB = 128
NSUB = CH // GSUB
U = 16


def _align8(v):
    return pl.multiple_of(v - lax.rem(v, 8), 8)


def _sc_call(xT, crow_pad, segs, kern, bias_pad):
    mesh = plsc.VectorSubcoreMesh(core_axis_name="c", subcore_axis_name="s")

    @functools.partial(
        pl.kernel,
        mesh=mesh,
        out_type=jax.ShapeDtypeStruct((NSEG_PAD, B), jnp.float32),
        scratch_types=[
            pltpu.VMEM((2, CH), jnp.int32),
            pltpu.VMEM((2, CH + 16, B), jnp.float32),
            pltpu.VMEM((2, FS), jnp.float32),
            pltpu.VMEM((CH,), jnp.int32),
            pltpu.VMEM((CH + 16, B), jnp.float32),
            pltpu.VMEM((CPW + 24,), jnp.int32),
            pltpu.VMEM((CPW + 16,), jnp.float32),
            pltpu.VMEM((CPW, B), jnp.float32),
            pltpu.SemaphoreType.DMA,
            pltpu.SemaphoreType.DMA,
            pltpu.SemaphoreType.DMA,
            pltpu.SemaphoreType.DMA,
        ],
        compiler_params=pltpu.CompilerParams(
            needs_layout_passes=False, use_tc_tiling_on_sc=False),
    )
    def k(xT_hbm, crow_hbm, segs_hbm, kern_hbm, bias_hbm, out_hbm,
          idx_v, rows_v, kcol_v, idx_s, rows_s, segs_v, bias_v, out_v,
          sem_g, sem_idx, sem_k, sem_s):
        wid = lax.axis_index("s") * NC + lax.axis_index("c")
        col0 = pl.multiple_of(wid * CPW, 8)
        pltpu.sync_copy(segs_hbm.at[pl.ds(col0, CPW + 8)],
                        segs_v.at[pl.ds(0, CPW + 8)])
        pltpu.sync_copy(bias_hbm.at[pl.ds(col0, CPW)],
                        bias_v.at[pl.ds(0, CPW)])
        zero16 = jnp.zeros((B,), jnp.float32)
        for r in range(16):
            rows_v[0, CH + r] = zero16
            rows_v[1, CH + r] = zero16
            rows_s[CH + r] = zero16

        iota16 = lax.iota(jnp.int32, 16)

        def kcol_src(lj):
            return kern_hbm.at[jnp.minimum(col0 + lj, N_OUT - 1)]

        def fire_kcol(lj, par):
            pltpu.async_copy(kcol_src(lj), kcol_v.at[par], sem_k)

        def wait_kcol(lj, par):
            pltpu.make_async_copy(kcol_src(lj), kcol_v.at[par], sem_k).wait()

        def fire_gathers(par, base, e, sem):
            for q in range(NSUB):
                @pl.when(base + q * GSUB < e + 15)
                def _():
                    pltpu.async_copy(
                        xT_hbm.at[idx_v.at[par, pl.ds(q * GSUB, GSUB)]],
                        rows_v.at[par, pl.ds(q * GSUB, GSUB)],
                        sem)

        def drain_gathers(par, base, e, sem):
            for q in range(NSUB):
                @pl.when(base + q * GSUB < e + 15)
                def _():
                    pltpu.make_async_copy(
                        xT_hbm.at[idx_v.at[par, pl.ds(q * GSUB, GSUB)]],
                        rows_v.at[par, pl.ds(q * GSUB, GSUB)],
                        sem).wait()

        def chunk_compute(load_row, par16, base, s, e, accs):
            c_lo = jnp.maximum(s - base, 0)
            c_hi = jnp.minimum(e - base, CH)
            ntrip = lax.div(jnp.maximum(c_hi - c_lo, 0) + U - 1, U)

            def conn_body(i, a):
                a = list(a)
                c0 = c_lo + i * U
                p0 = base + c0 - s
                pos16 = jnp.clip(p0 + iota16, 0, FS - 1)
                w16 = plsc.load_gather(kcol_v, [par16, pos16])
                w16 = jnp.where(iota16 < c_hi - c0, w16, 0.0)
                for u in range(U):
                    a[u % 4] = a[u % 4] + load_row(c0 + u) * w16[u]
                return tuple(a)

            return lax.fori_loop(0, ntrip, conn_body, accs)

        sv0 = segs_v[pl.ds(0, 16)]
        s0, e0, e1, e2_0 = sv0[0], sv0[1], sv0[2], sv0[3]
        ab0 = _align8(s0)
        ab1 = _align8(e0)
        pltpu.sync_copy(crow_hbm.at[pl.ds(ab0, CH)], idx_v.at[0])
        fire_gathers(0, ab0, e0, sem_g)
        fire_kcol(0, 0)
        pltpu.async_copy(crow_hbm.at[pl.ds(ab1, CH)], idx_v.at[1], sem_idx)

        def one_seg(lj, par, s, e, e2):
            par2 = 1 - par
            ab = _align8(s)
            ab2 = _align8(e)
            ab3 = _align8(e2)

            @pl.when(lj + 1 < CPW)
            def _():
                pltpu.make_async_copy(crow_hbm.at[pl.ds(ab2, CH)],
                                      idx_v.at[par2], sem_idx).wait()
            drain_gathers(par, ab, e, sem_g)
            @pl.when(lj + 1 < CPW)
            def _():
                fire_gathers(par2, ab2, e2, sem_g)
            @pl.when(lj + 2 < CPW)
            def _():
                pltpu.async_copy(crow_hbm.at[pl.ds(ab3, CH)],
                                 idx_v.at[par], sem_idx)
            wait_kcol(lj, par)

            @pl.when(lj + 1 < CPW)
            def _():
                fire_kcol(lj + 1, par2)

            par16 = jnp.full((16,), par, jnp.int32)

            z = jnp.zeros((B,), jnp.float32)
            acc = chunk_compute(lambda c: rows_v[par, c], par16, ab, s, e,
                                (z, z, z, z))
            span = e - ab
            nch = lax.div(span + CH - 1, CH)

            def extra_chunk(cc, a):
                cbase = pl.multiple_of(ab + cc * CH, 8)
                pltpu.sync_copy(crow_hbm.at[pl.ds(cbase, CH)], idx_s)
                for q in range(NSUB):
                    @pl.when(cbase + q * GSUB < e + 15)
                    def _():
                        pltpu.async_copy(
                            xT_hbm.at[idx_s.at[pl.ds(q * GSUB, GSUB)]],
                            rows_s.at[pl.ds(q * GSUB, GSUB)],
                            sem_s)
                for q in range(NSUB):
                    @pl.when(cbase + q * GSUB < e + 15)
                    def _():
                        pltpu.make_async_copy(
                            xT_hbm.at[idx_s.at[pl.ds(q * GSUB, GSUB)]],
                            rows_s.at[pl.ds(q * GSUB, GSUB)],
                            sem_s).wait()
                return chunk_compute(lambda c: rows_s[c], par16, cbase, s, e, a)

            acc = lax.fori_loop(1, nch, extra_chunk, acc)
            atot = (acc[0] + acc[1]) + (acc[2] + acc[3])

            lj16 = jnp.full((16,), lj, jnp.int32)
            bvec = plsc.load_gather(bias_v, [lj16])
            out_v[lj] = jnp.maximum(atot + bvec, 0.0)

        def seg_body2(t, carry):
            b0, b1, b2, b3 = carry
            lj = t * 2
            sv = segs_v[pl.ds(lj + 4, 16)]
            n0, n1 = sv[0], sv[1]
            one_seg(lj, 0, b0, b1, b2)
            one_seg(lj + 1, 1, b1, b2, b3)
            return (b2, b3, n0, n1)

        lax.fori_loop(0, CPW // 2, seg_body2, (s0, e0, e1, e2_0))
        pltpu.sync_copy(out_v, out_hbm.at[pl.ds(col0, CPW)])

    return k(xT, crow_pad, segs, kern, bias_pad)


def kernel(x, conn_row, conn_col, kernel, bias):
    xT = x.T
    crow_pad = jnp.concatenate(
        [conn_row, jnp.zeros((CH + 8,), jnp.int32)])
    segs = jnp.searchsorted(
        conn_col, jnp.arange(NSEG_PAD + 8, dtype=jnp.int32)).astype(jnp.int32)
    bias_pad = jnp.concatenate(
        [bias[:, 0], jnp.zeros((NSEG_PAD - N_OUT,), jnp.float32)])
    outT = _sc_call(xT, crow_pad, segs, kernel.T, bias_pad)
    return outT[:N_OUT].T[:, :, None]


# speedup vs baseline: 1.7634x; 1.7634x over previous
import functools
import jax
import jax.numpy as jnp
from jax import lax
from jax.experimental import pallas as pl
from jax.experimental.pallas import tpu as pltpu
from jax.experimental.pallas import tpu_sc as plsc

B = 16
N_IN = 100000
N_OUT = 10000
FS = 512

NC = 2
NS = 16
NW = NC * NS
NSEG_PAD = 10240
CPW = NSEG_PAD // NW
CH = 512
GSUB = 128
NSUB = CH // GSUB
U = 16


def _align8(v):
    return pl.multiple_of(v - lax.rem(v, 8), 8)


def _sc_call(xT, crow_pad, segs, kern, bias_pad):
    mesh = plsc.VectorSubcoreMesh(core_axis_name="c", subcore_axis_name="s")

    @functools.partial(
        pl.kernel,
        mesh=mesh,
        out_type=jax.ShapeDtypeStruct((NSEG_PAD, B), jnp.float32),
        scratch_types=[
            pltpu.VMEM((2, CH), jnp.int32),
            pltpu.VMEM((2, CH + 16, B), jnp.float32),
            pltpu.VMEM((2, FS), jnp.float32),
            pltpu.VMEM((CH,), jnp.int32),
            pltpu.VMEM((CH + 16, B), jnp.float32),
            pltpu.VMEM((CPW + 24,), jnp.int32),
            pltpu.VMEM((CPW + 16,), jnp.float32),
            pltpu.VMEM((CPW, B), jnp.float32),
            pltpu.SemaphoreType.DMA,
            pltpu.SemaphoreType.DMA,
            pltpu.SemaphoreType.DMA,
            pltpu.SemaphoreType.DMA,
        ],
        compiler_params=pltpu.CompilerParams(
            needs_layout_passes=False, use_tc_tiling_on_sc=False),
    )
    def k(xT_hbm, crow_hbm, segs_hbm, kern_hbm, bias_hbm, out_hbm,
          idx_v, rows_v, kcol_v, idx_s, rows_s, segs_v, bias_v, out_v,
          sem_g, sem_idx, sem_k, sem_s):
        wid = lax.axis_index("s") * NC + lax.axis_index("c")
        col0 = pl.multiple_of(wid * CPW, 8)
        pltpu.sync_copy(segs_hbm.at[pl.ds(col0, CPW + 8)],
                        segs_v.at[pl.ds(0, CPW + 8)])
        pltpu.sync_copy(bias_hbm.at[pl.ds(col0, CPW)],
                        bias_v.at[pl.ds(0, CPW)])
        zero16 = jnp.zeros((B,), jnp.float32)
        for r in range(16):
            rows_v[0, CH + r] = zero16
            rows_v[1, CH + r] = zero16
            rows_s[CH + r] = zero16

        iota16 = lax.iota(jnp.int32, 16)

        def kcol_src(lj):
            return kern_hbm.at[jnp.minimum(col0 + lj, N_OUT - 1)]

        def fire_kcol(lj, par):
            pltpu.async_copy(kcol_src(lj), kcol_v.at[par], sem_k)

        def wait_kcol(lj, par):
            pltpu.make_async_copy(kcol_src(lj), kcol_v.at[par], sem_k).wait()

        def fire_gathers(par, base, e, sem):
            for q in range(NSUB):
                @pl.when(base + q * GSUB < e + 15)
                def _():
                    pltpu.async_copy(
                        xT_hbm.at[idx_v.at[par, pl.ds(q * GSUB, GSUB)]],
                        rows_v.at[par, pl.ds(q * GSUB, GSUB)],
                        sem)

        def drain_gathers(par, base, e, sem):
            for q in range(NSUB):
                @pl.when(base + q * GSUB < e + 15)
                def _():
                    pltpu.make_async_copy(
                        xT_hbm.at[idx_v.at[par, pl.ds(q * GSUB, GSUB)]],
                        rows_v.at[par, pl.ds(q * GSUB, GSUB)],
                        sem).wait()

        def chunk_compute(load_row, par16, base, s, e, accs):
            c_lo = jnp.maximum(s - base, 0)
            c_hi = jnp.minimum(e - base, CH)
            ntrip = lax.div(jnp.maximum(c_hi - c_lo, 0) + U - 1, U)

            def conn_body(i, a):
                a = list(a)
                c0 = c_lo + i * U
                p0 = base + c0 - s
                pos16 = jnp.clip(p0 + iota16, 0, FS - 1)
                w16 = plsc.load_gather(kcol_v, [par16, pos16])
                w16 = jnp.where(iota16 < c_hi - c0, w16, 0.0)
                for u in range(U):
                    a[u % 4] = a[u % 4] + load_row(c0 + u) * w16[u]
                return tuple(a)

            return lax.fori_loop(0, ntrip, conn_body, accs)

        sv0 = segs_v[pl.ds(0, 16)]
        s0, e0, e1, e2_0 = sv0[0], sv0[1], sv0[2], sv0[3]
        ab0 = _align8(s0)
        ab1 = _align8(e0)
        pltpu.sync_copy(crow_hbm.at[pl.ds(ab0, CH)], idx_v.at[0])
        fire_gathers(0, ab0, e0, sem_g)
        fire_kcol(0, 0)
        pltpu.async_copy(crow_hbm.at[pl.ds(ab1, CH)], idx_v.at[1], sem_idx)

        def one_seg(lj, par, s, e, e2):
            par2 = 1 - par
            ab = _align8(s)
            ab2 = _align8(e)
            ab3 = _align8(e2)

            @pl.when(lj + 1 < CPW)
            def _():
                pltpu.make_async_copy(crow_hbm.at[pl.ds(ab2, CH)],
                                      idx_v.at[par2], sem_idx).wait()
            drain_gathers(par, ab, e, sem_g)
            @pl.when(lj + 1 < CPW)
            def _():
                fire_gathers(par2, ab2, e2, sem_g)
            @pl.when(lj + 2 < CPW)
            def _():
                pltpu.async_copy(crow_hbm.at[pl.ds(ab3, CH)],
                                 idx_v.at[par], sem_idx)
            wait_kcol(lj, par)

            @pl.when(lj + 1 < CPW)
            def _():
                fire_kcol(lj + 1, par2)

            par16 = jnp.full((16,), par, jnp.int32)

            z = jnp.zeros((B,), jnp.float32)
            acc = chunk_compute(lambda c: rows_v[par, c], par16, ab, s, e,
                                (z, z, z, z))
            span = e - ab
            nch = lax.div(span + CH - 1, CH)

            def extra_chunk(cc, a):
                cbase = pl.multiple_of(ab + cc * CH, 8)
                pltpu.sync_copy(crow_hbm.at[pl.ds(cbase, CH)], idx_s)
                for q in range(NSUB):
                    @pl.when(cbase + q * GSUB < e + 15)
                    def _():
                        pltpu.async_copy(
                            xT_hbm.at[idx_s.at[pl.ds(q * GSUB, GSUB)]],
                            rows_s.at[pl.ds(q * GSUB, GSUB)],
                            sem_s)
                for q in range(NSUB):
                    @pl.when(cbase + q * GSUB < e + 15)
                    def _():
                        pltpu.make_async_copy(
                            xT_hbm.at[idx_s.at[pl.ds(q * GSUB, GSUB)]],
                            rows_s.at[pl.ds(q * GSUB, GSUB)],
                            sem_s).wait()
                return chunk_compute(lambda c: rows_s[c], par16, cbase, s, e, a)

            acc = lax.fori_loop(1, nch, extra_chunk, acc)
            atot = (acc[0] + acc[1]) + (acc[2] + acc[3])

            lj16 = jnp.full((16,), lj, jnp.int32)
            bvec = plsc.load_gather(bias_v, [lj16])
            out_v[lj] = jnp.maximum(atot + bvec, 0.0)

        def seg_body2(t, carry):
            b0, b1, b2, b3 = carry
            lj = t * 2
            sv = segs_v[pl.ds(lj + 4, 16)]
            n0, n1 = sv[0], sv[1]
            one_seg(lj, 0, b0, b1, b2)
            one_seg(lj + 1, 1, b1, b2, b3)
            return (b2, b3, n0, n1)

        lax.fori_loop(0, CPW // 2, seg_body2, (s0, e0, e1, e2_0))
        pltpu.sync_copy(out_v, out_hbm.at[pl.ds(col0, CPW)])

    return k(xT, crow_pad, segs, kern, bias_pad)


def kernel(x, conn_row, conn_col, kernel, bias):
    xT = x.T
    crow_pad = jnp.concatenate(
        [conn_row, jnp.zeros((CH + 8,), jnp.int32)])
    hist = jnp.zeros((NSEG_PAD + 8,), jnp.int32).at[conn_col].add(1)
    csum = jnp.cumsum(hist)
    segs = jnp.concatenate([jnp.zeros((1,), jnp.int32), csum[:-1]])
    bias_pad = jnp.concatenate(
        [bias[:, 0], jnp.zeros((NSEG_PAD - N_OUT,), jnp.float32)])
    outT = _sc_call(xT, crow_pad, segs, kernel.T, bias_pad)
    return outT[:N_OUT].T[:, :, None]
